# baseline (speedup 1.0000x reference)
import functools

import jax
import jax.numpy as jnp
from jax import lax
from jax.experimental import pallas as pl
from jax.experimental.pallas import tpu as pltpu
from jax.experimental.pallas import tpu_sc as plsc

NC = 2
NS = 16
NW = NC * NS
BATCH = 16384
HIDDEN = 64
CHUNK = 128
B_PER_W = BATCH // NW
NCHUNK = B_PER_W // CHUNK
PAIR = 2 * HIDDEN

NTAB = 1000000
HB = 512
NHB = NTAB // HB
TGRID = 977
OUT_ROWS = TGRID * HB
RSB = 1024
RSGRID = (NTAB + RSB - 1) // RSB
RS_LEN = RSGRID * RSB


def _transpose_body(inA, inB, out_ref):
    out_ref[:, 0:HIDDEN] = inA[...].T
    out_ref[:, HIDDEN:PAIR] = inB[...].T


def _transpose_pairs(tT):
    return pl.pallas_call(
        _transpose_body,
        grid=(TGRID,),
        in_specs=[
            pl.BlockSpec((HIDDEN, HB), lambda j: (0, jnp.minimum(2 * j, NHB))),
            pl.BlockSpec((HIDDEN, HB),
                         lambda j: (0, jnp.minimum(2 * j + 1, NHB))),
        ],
        out_specs=pl.BlockSpec((HB, PAIR), lambda j: (j, 0)),
        out_shape=jax.ShapeDtypeStruct((OUT_ROWS, PAIR), jnp.float32),
    )(tT, tT)


def _rowsum_body(in_ref, out_ref):
    out_ref[...] = jnp.sum(in_ref[...], axis=0)


def _rowsum(tT):
    return pl.pallas_call(
        _rowsum_body,
        grid=(RSGRID,),
        in_specs=[pl.BlockSpec((HIDDEN, RSB), lambda j: (0, j))],
        out_specs=pl.BlockSpec((RSB,), lambda j: (j,)),
        out_shape=jax.ShapeDtypeStruct((RS_LEN,), jnp.float32),
    )(tT)


def _sc_body(users_ref, items_ref, uf_hbm, if_hbm, rsu_hbm, rsi_hbm, out_hbm,
             uidx_v, iidx_v, up_v, ip_v, uf_v, if_v, rsu_v, rsi_v, out_v, sem):
    wid = lax.axis_index("s") * NC + lax.axis_index("c")
    base = wid * B_PER_W
    row0 = wid * NCHUNK

    pltpu.sync_copy(users_ref.at[pl.ds(row0, NCHUNK)], uidx_v)
    pltpu.sync_copy(items_ref.at[pl.ds(row0, NCHUNK)], iidx_v)

    for c in range(NCHUNK):
        for g in range(CHUNK // 16):
            s = pl.ds(g * 16, 16)
            u = uidx_v[c, s]
            i = iidx_v[c, s]
            up_v[c, s] = ((u >> 10) << 9) | (u & 511)
            ip_v[c, s] = ((i >> 10) << 9) | (i & 511)

    lanes = jax.lax.iota(jnp.int32, 16)
    last_lane = lanes == 15

    for c in range(NCHUNK):
        cp0 = pltpu.async_copy(uf_hbm.at[up_v.at[c]], uf_v, sem)
        cp1 = pltpu.async_copy(if_hbm.at[ip_v.at[c]], if_v, sem)
        cp2 = pltpu.async_copy(rsu_hbm.at[uidx_v.at[c]], rsu_v.at[c], sem)
        cp3 = pltpu.async_copy(rsi_hbm.at[iidx_v.at[c]], rsi_v.at[c], sem)
        cp0.wait()
        cp1.wait()
        cp2.wait()
        cp3.wait()

        def group(g, _):
            s16 = pl.ds(g * 16, 16)
            ubase = ((uidx_v[c, s16] >> 9) & 1) * HIDDEN
            ibase = ((iidx_v[c, s16] >> 9) & 1) * HIDDEN
            for l in range(16):
                e = g * 16 + l
                bu = ubase[l]
                bi = ibase[l]
                acc = None
                for j in range(HIDDEN // 16):
                    su = pl.ds(bu + j * 16, 16)
                    si = pl.ds(bi + j * 16, 16)
                    t = uf_v[e, su] * if_v[e, si]
                    acc = t if acc is None else acc + t
                sums = plsc.cumsum(acc)
                plsc.store_scatter(out_v,
                                   [jnp.full((16,), c * CHUNK + e, jnp.int32)],
                                   sums, mask=last_lane)
            return 0

        lax.fori_loop(0, CHUNK // 16, group, 0)

        for g in range(CHUNK // 16):
            s16 = pl.ds(g * 16, 16)
            so = pl.ds(c * CHUNK + g * 16, 16)
            out_v[so] = out_v[so] + rsu_v[c, s16] + rsi_v[c, s16]

    pltpu.sync_copy(out_v, out_hbm.at[pl.ds(base, B_PER_W)])


@functools.partial(jax.jit, static_argnames=())
def kernel(users, items, user_factors, item_factors, user_biases, item_biases):
    uf2 = _transpose_pairs(user_factors.T)
    if2 = _transpose_pairs(item_factors.T)
    rsu = _rowsum(user_biases.T)
    rsi = _rowsum(item_biases.T)

    mesh = plsc.VectorSubcoreMesh(
        core_axis_name="c", subcore_axis_name="s",
        num_cores=NC, num_subcores=NS)
    f = pl.kernel(
        _sc_body,
        out_type=jax.ShapeDtypeStruct((BATCH,), jnp.float32),
        mesh=mesh,
        compiler_params=pltpu.CompilerParams(needs_layout_passes=False,
                                             use_tc_tiling_on_sc=True),
        scratch_types=[
            pltpu.VMEM((NCHUNK, CHUNK), jnp.int32),
            pltpu.VMEM((NCHUNK, CHUNK), jnp.int32),
            pltpu.VMEM((NCHUNK, CHUNK), jnp.int32),
            pltpu.VMEM((NCHUNK, CHUNK), jnp.int32),
            pltpu.VMEM((CHUNK, PAIR), jnp.float32),
            pltpu.VMEM((CHUNK, PAIR), jnp.float32),
            pltpu.VMEM((NCHUNK, CHUNK), jnp.float32),
            pltpu.VMEM((NCHUNK, CHUNK), jnp.float32),
            pltpu.VMEM((B_PER_W,), jnp.float32),
            pltpu.SemaphoreType.DMA,
        ],
    )
    out = f(users.reshape(BATCH // CHUNK, CHUNK),
            items.reshape(BATCH // CHUNK, CHUNK),
            uf2, if2, rsu, rsi)
    return out.reshape(BATCH, 1)


# speedup vs baseline: 1.3578x; 1.3578x over previous
import functools

import jax
import jax.numpy as jnp
from jax import lax
from jax.experimental import pallas as pl
from jax.experimental.pallas import tpu as pltpu
from jax.experimental.pallas import tpu_sc as plsc

NC = 2
NS = 16
NW = NC * NS
BATCH = 16384
HIDDEN = 64
CHUNK = 32
B_PER_W = BATCH // NW
NCHUNK = B_PER_W // CHUNK

NTAB = 1000000
RSB = 1024
RSGRID = (NTAB + RSB - 1) // RSB
RS_LEN = RSGRID * RSB


def _rowsum_body(in_ref, out_ref):
    out_ref[...] = jnp.sum(in_ref[...], axis=0)


def _rowsum(tT):
    return pl.pallas_call(
        _rowsum_body,
        grid=(RSGRID,),
        in_specs=[pl.BlockSpec((HIDDEN, RSB), lambda j: (0, j))],
        out_specs=pl.BlockSpec((RSB,), lambda j: (j,)),
        out_shape=jax.ShapeDtypeStruct((RS_LEN,), jnp.float32),
    )(tT)


def _sc_body(users_ref, items_ref, uf_hbm, if_hbm, rs_hbm, out_hbm,
             uidx_v, iidx_v, ridx_v, ufs_v, ifs_v, rsb_v, out_v, sem):
    wid = lax.axis_index("s") * NC + lax.axis_index("c")
    base = wid * B_PER_W

    pltpu.sync_copy(users_ref.at[pl.ds(base, B_PER_W)], uidx_v)
    pltpu.sync_copy(items_ref.at[pl.ds(base, B_PER_W)], iidx_v)

    lanes = jax.lax.iota(jnp.int32, 16)
    last_lane = lanes == 15

    def chunk(c, _):
        c0 = c * CHUNK
        def roff(g, _):
            s = pl.ds(g * 16, 16)
            ridx_v[s] = uidx_v[pl.ds(c0 + g * 16, 16)]
            ridx_v[pl.ds(CHUNK + g * 16, 16)] = (
                iidx_v[pl.ds(c0 + g * 16, 16)] + RS_LEN)
            return 0

        lax.fori_loop(0, CHUNK // 16, roff, 0)
        pltpu.async_copy(rs_hbm.at[ridx_v], rsb_v, sem)

        def fire(g, _):
            s16 = pl.ds(c0 + g * 16, 16)
            uvals = uidx_v[s16]
            ivals = iidx_v[s16]
            for l in range(16):
                e = g * 16 + l
                u = uvals[l]
                i = ivals[l]
                pltpu.async_copy(uf_hbm.at[pl.ds((u >> 3) * 8, 8)],
                                 ufs_v.at[e], sem)
                pltpu.async_copy(if_hbm.at[pl.ds((i >> 3) * 8, 8)],
                                 ifs_v.at[e], sem)
            return 0

        lax.fori_loop(0, CHUNK // 16, fire, 0)

        dummy = uf_hbm.at[pl.ds(0, 8 * CHUNK)].reshape(CHUNK, 8, HIDDEN)
        pltpu.make_async_copy(dummy, ufs_v, sem).wait()
        pltpu.make_async_copy(dummy, ifs_v, sem).wait()
        pltpu.make_async_copy(rs_hbm.at[pl.ds(0, 2 * CHUNK)], rsb_v, sem).wait()

        def compute(g, _):
            s16 = pl.ds(c0 + g * 16, 16)
            usub = uidx_v[s16] & 7
            isub = iidx_v[s16] & 7
            for l in range(16):
                e = g * 16 + l
                ur = usub[l]
                ir = isub[l]
                acc = None
                for j in range(HIDDEN // 16):
                    sj = pl.ds(j * 16, 16)
                    t = ufs_v[e, ur, sj] * ifs_v[e, ir, sj]
                    acc = t if acc is None else acc + t
                sums = plsc.cumsum(acc)
                plsc.store_scatter(out_v,
                                   [jnp.full((16,), c0 + e, jnp.int32)],
                                   sums, mask=last_lane)
            return 0

        lax.fori_loop(0, CHUNK // 16, compute, 0)

        def bias(g, _):
            so = pl.ds(c0 + g * 16, 16)
            sg = pl.ds(g * 16, 16)
            sg2 = pl.ds(CHUNK + g * 16, 16)
            out_v[so] = out_v[so] + rsb_v[sg] + rsb_v[sg2]
            return 0

        lax.fori_loop(0, CHUNK // 16, bias, 0)
        return 0

    lax.fori_loop(0, NCHUNK, chunk, 0)

    pltpu.sync_copy(out_v, out_hbm.at[pl.ds(base, B_PER_W)])


@functools.partial(jax.jit, static_argnames=())
def kernel(users, items, user_factors, item_factors, user_biases, item_biases):
    rs_all = jnp.concatenate([_rowsum(user_biases.T),
                              _rowsum(item_biases.T)])

    mesh = plsc.VectorSubcoreMesh(
        core_axis_name="c", subcore_axis_name="s",
        num_cores=NC, num_subcores=NS)
    f = pl.kernel(
        _sc_body,
        out_type=jax.ShapeDtypeStruct((BATCH,), jnp.float32),
        mesh=mesh,
        compiler_params=pltpu.CompilerParams(needs_layout_passes=False,
                                             use_tc_tiling_on_sc=True),
        scratch_types=[
            pltpu.VMEM((B_PER_W,), jnp.int32),
            pltpu.VMEM((B_PER_W,), jnp.int32),
            pltpu.VMEM((2 * CHUNK,), jnp.int32),
            pltpu.VMEM((CHUNK, 8, HIDDEN), jnp.float32),
            pltpu.VMEM((CHUNK, 8, HIDDEN), jnp.float32),
            pltpu.VMEM((2 * CHUNK,), jnp.float32),
            pltpu.VMEM((B_PER_W,), jnp.float32),
            pltpu.SemaphoreType.DMA,
        ],
    )
    out = f(users, items, user_factors, item_factors, rs_all)
    return out.reshape(BATCH, 1)
